# baseline (speedup 1.0000x reference)
import functools

import jax
import jax.numpy as jnp
from jax import lax
from jax.experimental import pallas as pl
from jax.experimental.pallas import tpu as pltpu
from jax.experimental.pallas import tpu_sc as plsc

L = 16
NW = 4


def _sc_body(x0_h, w1s_h, b1s_h, w2s_h, b2s_h, w1c_h, b1c_h, w2c_h, b2c_h,
             out_h, wa, ba, wb, bb, v_scr, h_scr, p_scr, q_scr, o_scr, shared,
             *, n, d):
    cid = lax.axis_index("c")
    sid = lax.axis_index("s")
    dh = 2 * d // NW
    nc = d // L
    nh = dh // L
    is_spread = cid == 0
    is_worker = sid < NW

    @pl.when(jnp.logical_and(is_worker, is_spread))
    def _():
        pltpu.sync_copy(x0_h, v_scr)
        pltpu.sync_copy(w1s_h.at[sid], wa)
        pltpu.sync_copy(b1s_h.at[sid], ba)
        pltpu.sync_copy(w2s_h.at[sid], wb)
        pltpu.sync_copy(b2s_h, bb)

    @pl.when(jnp.logical_and(is_worker, jnp.logical_not(is_spread)))
    def _():
        pltpu.sync_copy(w1c_h.at[sid], wa)
        pltpu.sync_copy(b1c_h.at[sid], ba)
        pltpu.sync_copy(w2c_h.at[sid], wb)
        pltpu.sync_copy(b2c_h, bb)

    @pl.when(jnp.logical_and(is_worker, sid > 0))
    def _():
        for c in range(nc):
            bb[pl.ds(c * L, L)] = jnp.zeros((L,), jnp.float32)

    dirn = 1 - 2 * cid
    pos0 = 1 + (n - 2) * cid

    def step(ii, pos):
        @pl.when(is_worker)
        def _():
            @pl.when(pos != ii)
            def _():
                for c in range(nc):
                    v_scr[pl.ds(c * L, L)] = jnp.zeros((L,), jnp.float32)

            def body1(jc, acc):
                vchunk = v_scr[pl.ds(jc * L, L)]
                for jl in range(L):
                    bj = jnp.full((L,), vchunk[jl], jnp.float32)
                    acc = tuple(acc[o] + bj * wa[jc * L + jl, pl.ds(o * L, L)]
                                for o in range(nh))
                return acc

            h = lax.fori_loop(0, nc, body1,
                              tuple(ba[pl.ds(o * L, L)] for o in range(nh)))
            for o in range(nh):
                h_scr[pl.ds(o * L, L)] = jnp.maximum(h[o], 0.0)

            def body2(jc, acc):
                vchunk = h_scr[pl.ds(jc * L, L)]
                for jl in range(L):
                    bj = jnp.full((L,), vchunk[jl], jnp.float32)
                    acc = tuple(acc[o] + bj * wb[jc * L + jl, pl.ds(o * L, L)]
                                for o in range(nc))
                return acc

            part = lax.fori_loop(0, nh, body2,
                                 tuple(bb[pl.ds(o * L, L)] for o in range(nc)))
            for o in range(nc):
                p_scr[pl.ds(o * L, L)] = part[o]
            pltpu.sync_copy(p_scr, shared.at[ii % 2, sid])

        plsc.subcore_barrier()

        @pl.when(is_worker)
        def _():
            for c in range(nc):
                v_scr[pl.ds(c * L, L)] = p_scr[pl.ds(c * L, L)]
            for k in range(1, NW):
                pltpu.sync_copy(shared.at[ii % 2, (sid + k) % NW], q_scr)
                for c in range(nc):
                    v_scr[pl.ds(c * L, L)] = (v_scr[pl.ds(c * L, L)]
                                              + q_scr[pl.ds(c * L, L)])

        return ii + dirn

    pos = lax.fori_loop(1, n, step, pos0)

    @pl.when(jnp.logical_and(cid == 1, sid == 0))
    def _():
        for c in range(nc):
            o_scr[pl.ds(c * L, L)] = v_scr[pl.ds(c * L, L)]

        @pl.when(pos != 0)
        def _():
            for c in range(nc):
                o_scr[pl.ds(c * L, L)] = jnp.zeros((L,), jnp.float32)

        pltpu.sync_copy(o_scr, out_h)


def kernel(x, num_node, edge_index, W1s, b1s, W2s, b2s, W1c, b1c, W2c, b2c):
    del num_node, edge_index
    n = x.shape[0]
    assert n > 2
    d = W2s.shape[0]
    assert W2c.shape[0] == d and W1s.shape[0] == 2 * d and W1c.shape[0] == 2 * d
    dh = 2 * d // NW
    x0 = jnp.pad(x[0, :], (0, d - x.shape[1]))

    def split1(w1):
        return w1.T.reshape(d, NW, dh).transpose(1, 0, 2)

    def split2(w2):
        return w2.T.reshape(NW, dh, d)

    mesh = plsc.VectorSubcoreMesh(core_axis_name="c", subcore_axis_name="s")
    body = functools.partial(_sc_body, n=n, d=d)
    run = pl.kernel(
        body,
        out_type=jax.ShapeDtypeStruct((d,), jnp.float32),
        mesh=mesh,
        scratch_types=[
            pltpu.VMEM((d, dh), jnp.float32),
            pltpu.VMEM((dh,), jnp.float32),
            pltpu.VMEM((dh, d), jnp.float32),
            pltpu.VMEM((d,), jnp.float32),
            pltpu.VMEM((d,), jnp.float32),
            pltpu.VMEM((dh,), jnp.float32),
            pltpu.VMEM((d,), jnp.float32),
            pltpu.VMEM((d,), jnp.float32),
            pltpu.VMEM((d,), jnp.float32),
            pltpu.VMEM_SHARED((2, NW, d), jnp.float32),
        ],
    )
    return run(x0,
               split1(W1s), b1s.reshape(NW, dh), split2(W2s), b2s,
               split1(W1c), b1c.reshape(NW, dh), split2(W2c), b2c)


# speedup vs baseline: 1.4189x; 1.4189x over previous
import functools

import jax
import jax.numpy as jnp
from jax import lax
from jax.experimental import pallas as pl
from jax.experimental.pallas import tpu as pltpu
from jax.experimental.pallas import tpu_sc as plsc

L = 16
NW = 4


def _sc_body(x0_h, w1s_h, b1s_h, w2s_h, b2s_h, w1c_h, b1c_h, w2c_h, b2c_h,
             out_h, wa, ba, wb, bb, v_scr, h_scr, p_scr, q_scr, o_scr, shared,
             *, n, d):
    cid = lax.axis_index("c")
    sid = lax.axis_index("s")
    dh = 2 * d // NW
    nc = d // L
    nh = dh // L
    is_spread = cid == 0
    is_worker = sid < NW

    @pl.when(jnp.logical_and(is_worker, is_spread))
    def _():
        pltpu.sync_copy(x0_h, v_scr)
        pltpu.sync_copy(w1s_h.at[sid], wa)
        pltpu.sync_copy(b1s_h.at[sid], ba)
        pltpu.sync_copy(w2s_h.at[sid], wb)
        pltpu.sync_copy(b2s_h, bb)

    @pl.when(jnp.logical_and(is_worker, jnp.logical_not(is_spread)))
    def _():
        pltpu.sync_copy(w1c_h.at[sid], wa)
        pltpu.sync_copy(b1c_h.at[sid], ba)
        pltpu.sync_copy(w2c_h.at[sid], wb)
        pltpu.sync_copy(b2c_h, bb)

    @pl.when(jnp.logical_and(is_worker, sid > 0))
    def _():
        for c in range(nc):
            bb[pl.ds(c * L, L)] = jnp.zeros((L,), jnp.float32)

    dirn = 1 - 2 * cid
    pos0 = 1 + (n - 2) * cid

    def step(ii, pos):
        @pl.when(is_worker)
        def _():
            @pl.when(pos != ii)
            def _():
                for c in range(nc):
                    v_scr[pl.ds(c * L, L)] = jnp.zeros((L,), jnp.float32)

            def body1(jc, acc):
                vchunk = v_scr[pl.ds(jc * L, L)]
                for jl in range(L):
                    bj = jnp.full((L,), vchunk[jl], jnp.float32)
                    acc = tuple(acc[o] + bj * wa[jc * L + jl, pl.ds(o * L, L)]
                                for o in range(nh))
                return acc

            h = lax.fori_loop(0, nc, body1,
                              tuple(ba[pl.ds(o * L, L)] for o in range(nh)))
            for o in range(nh):
                h_scr[pl.ds(o * L, L)] = jnp.maximum(h[o], 0.0)

            def body2(jc, acc):
                vchunk = h_scr[pl.ds(jc * L, L)]
                for jl in range(L):
                    bj = jnp.full((L,), vchunk[jl], jnp.float32)
                    acc = tuple(acc[o] + bj * wb[jc * L + jl, pl.ds(o * L, L)]
                                for o in range(nc))
                return acc

            part = lax.fori_loop(0, nh, body2,
                                 tuple(bb[pl.ds(o * L, L)] for o in range(nc)))
            for o in range(nc):
                p_scr[pl.ds(o * L, L)] = part[o]
            pltpu.sync_copy(p_scr, shared.at[ii % 2, sid])

        plsc.subcore_barrier()

        @pl.when(is_worker)
        def _():
            pltpu.sync_copy(shared.at[ii % 2], q_scr)
            for c in range(nc):
                acc = q_scr[0, pl.ds(c * L, L)]
                for k in range(1, NW):
                    acc = acc + q_scr[k, pl.ds(c * L, L)]
                v_scr[pl.ds(c * L, L)] = acc

        return ii + dirn

    pos = lax.fori_loop(1, n, step, pos0)

    @pl.when(jnp.logical_and(cid == 1, sid == 0))
    def _():
        for c in range(nc):
            o_scr[pl.ds(c * L, L)] = v_scr[pl.ds(c * L, L)]

        @pl.when(pos != 0)
        def _():
            for c in range(nc):
                o_scr[pl.ds(c * L, L)] = jnp.zeros((L,), jnp.float32)

        pltpu.sync_copy(o_scr, out_h)


def kernel(x, num_node, edge_index, W1s, b1s, W2s, b2s, W1c, b1c, W2c, b2c):
    del num_node, edge_index
    n = x.shape[0]
    assert n > 2
    d = W2s.shape[0]
    assert W2c.shape[0] == d and W1s.shape[0] == 2 * d and W1c.shape[0] == 2 * d
    dh = 2 * d // NW
    x0 = jnp.pad(x[0, :], (0, d - x.shape[1]))

    def split1(w1):
        return w1.T.reshape(d, NW, dh).transpose(1, 0, 2)

    def split2(w2):
        return w2.T.reshape(NW, dh, d)

    mesh = plsc.VectorSubcoreMesh(core_axis_name="c", subcore_axis_name="s")
    body = functools.partial(_sc_body, n=n, d=d)
    run = pl.kernel(
        body,
        out_type=jax.ShapeDtypeStruct((d,), jnp.float32),
        mesh=mesh,
        scratch_types=[
            pltpu.VMEM((d, dh), jnp.float32),
            pltpu.VMEM((dh,), jnp.float32),
            pltpu.VMEM((dh, d), jnp.float32),
            pltpu.VMEM((d,), jnp.float32),
            pltpu.VMEM((d,), jnp.float32),
            pltpu.VMEM((dh,), jnp.float32),
            pltpu.VMEM((d,), jnp.float32),
            pltpu.VMEM((NW, d), jnp.float32),
            pltpu.VMEM((d,), jnp.float32),
            pltpu.VMEM_SHARED((2, NW, d), jnp.float32),
        ],
    )
    return run(x0,
               split1(W1s), b1s.reshape(NW, dh), split2(W2s), b2s,
               split1(W1c), b1c.reshape(NW, dh), split2(W2c), b2c)


# speedup vs baseline: 1.5172x; 1.0693x over previous
import functools

import jax
import jax.numpy as jnp
from jax import lax
from jax.experimental import pallas as pl
from jax.experimental.pallas import tpu as pltpu
from jax.experimental.pallas import tpu_sc as plsc

L = 16
NW = 8


def _sc_body(x0_h, w1s_h, b1s_h, w2s_h, b2s_h, w1c_h, b1c_h, w2c_h, b2c_h,
             out_h, wa, ba, wb, bb, v_scr, h_scr, p_scr, q_scr, o_scr, shared,
             *, n, d):
    cid = lax.axis_index("c")
    sid = lax.axis_index("s")
    dh = 2 * d // NW
    nc = d // L
    nh = dh // L
    is_spread = cid == 0
    is_worker = sid < NW

    @pl.when(jnp.logical_and(is_worker, is_spread))
    def _():
        pltpu.sync_copy(x0_h, v_scr)
        pltpu.sync_copy(w1s_h.at[sid], wa)
        pltpu.sync_copy(b1s_h.at[sid], ba)
        pltpu.sync_copy(w2s_h.at[sid], wb)
        pltpu.sync_copy(b2s_h, bb)

    @pl.when(jnp.logical_and(is_worker, jnp.logical_not(is_spread)))
    def _():
        pltpu.sync_copy(w1c_h.at[sid], wa)
        pltpu.sync_copy(b1c_h.at[sid], ba)
        pltpu.sync_copy(w2c_h.at[sid], wb)
        pltpu.sync_copy(b2c_h, bb)

    @pl.when(jnp.logical_and(is_worker, sid > 0))
    def _():
        for c in range(nc):
            bb[pl.ds(c * L, L)] = jnp.zeros((L,), jnp.float32)

    dirn = 1 - 2 * cid
    pos0 = 1 + (n - 2) * cid

    def step(ii, pos):
        @pl.when(is_worker)
        def _():
            @pl.when(pos != ii)
            def _():
                for c in range(nc):
                    v_scr[pl.ds(c * L, L)] = jnp.zeros((L,), jnp.float32)

            def body1(jc, acc):
                vchunk = v_scr[pl.ds(jc * L, L)]
                for jl in range(L):
                    bj = jnp.full((L,), vchunk[jl], jnp.float32)
                    acc = tuple(acc[o] + bj * wa[jc * L + jl, pl.ds(o * L, L)]
                                for o in range(nh))
                return acc

            h = lax.fori_loop(0, nc, body1,
                              tuple(ba[pl.ds(o * L, L)] for o in range(nh)))
            for o in range(nh):
                h_scr[pl.ds(o * L, L)] = jnp.maximum(h[o], 0.0)

            def body2(jc, acc):
                vchunk = h_scr[pl.ds(jc * L, L)]
                for jl in range(L):
                    bj = jnp.full((L,), vchunk[jl], jnp.float32)
                    acc = tuple(acc[o] + bj * wb[jc * L + jl, pl.ds(o * L, L)]
                                for o in range(nc))
                return acc

            part = lax.fori_loop(0, nh, body2,
                                 tuple(bb[pl.ds(o * L, L)] for o in range(nc)))
            for o in range(nc):
                p_scr[pl.ds(o * L, L)] = part[o]
            pltpu.sync_copy(p_scr, shared.at[ii % 2, sid])

        plsc.subcore_barrier()

        @pl.when(is_worker)
        def _():
            pltpu.sync_copy(shared.at[ii % 2], q_scr)
            for c in range(nc):
                acc = q_scr[0, pl.ds(c * L, L)]
                for k in range(1, NW):
                    acc = acc + q_scr[k, pl.ds(c * L, L)]
                v_scr[pl.ds(c * L, L)] = acc

        return ii + dirn

    pos = lax.fori_loop(1, n, step, pos0)

    @pl.when(jnp.logical_and(cid == 1, sid == 0))
    def _():
        for c in range(nc):
            o_scr[pl.ds(c * L, L)] = v_scr[pl.ds(c * L, L)]

        @pl.when(pos != 0)
        def _():
            for c in range(nc):
                o_scr[pl.ds(c * L, L)] = jnp.zeros((L,), jnp.float32)

        pltpu.sync_copy(o_scr, out_h)


def kernel(x, num_node, edge_index, W1s, b1s, W2s, b2s, W1c, b1c, W2c, b2c):
    del num_node, edge_index
    n = x.shape[0]
    assert n > 2
    d = W2s.shape[0]
    assert W2c.shape[0] == d and W1s.shape[0] == 2 * d and W1c.shape[0] == 2 * d
    dh = 2 * d // NW
    x0 = jnp.pad(x[0, :], (0, d - x.shape[1]))

    def split1(w1):
        return w1.T.reshape(d, NW, dh).transpose(1, 0, 2)

    def split2(w2):
        return w2.T.reshape(NW, dh, d)

    mesh = plsc.VectorSubcoreMesh(core_axis_name="c", subcore_axis_name="s")
    body = functools.partial(_sc_body, n=n, d=d)
    run = pl.kernel(
        body,
        out_type=jax.ShapeDtypeStruct((d,), jnp.float32),
        mesh=mesh,
        scratch_types=[
            pltpu.VMEM((d, dh), jnp.float32),
            pltpu.VMEM((dh,), jnp.float32),
            pltpu.VMEM((dh, d), jnp.float32),
            pltpu.VMEM((d,), jnp.float32),
            pltpu.VMEM((d,), jnp.float32),
            pltpu.VMEM((dh,), jnp.float32),
            pltpu.VMEM((d,), jnp.float32),
            pltpu.VMEM((NW, d), jnp.float32),
            pltpu.VMEM((d,), jnp.float32),
            pltpu.VMEM_SHARED((2, NW, d), jnp.float32),
        ],
    )
    return run(x0,
               split1(W1s), b1s.reshape(NW, dh), split2(W2s), b2s,
               split1(W1c), b1c.reshape(NW, dh), split2(W2c), b2c)
